# baseline (speedup 1.0000x reference)
import functools

import jax
import jax.numpy as jnp
from jax import lax
from jax.experimental import pallas as pl
from jax.experimental.pallas import tpu as pltpu
from jax.experimental.pallas import tpu_sc as plsc

B = 16384
DIM = 32
NC = 2
NS = 16
NW = NC * NS
BPW = B // NW
CH = 128
NCH = BPW // CH
L = 16


def _body(user_h, item_h, ut_h, it_h, but_h, bit_h, gb_h, out_h,
          uidx, iidx, urows, irows, bu, bi, gbv, outv, sem):
    cid = lax.axis_index("c")
    sid = lax.axis_index("s")
    wid = sid * NC + cid
    base = wid * BPW

    cps = []
    for j in range(NCH):
        cps.append(pltpu.async_copy(
            user_h.at[pl.ds(base + j * CH, CH)], uidx.at[j], sem))
        cps.append(pltpu.async_copy(
            item_h.at[pl.ds(base + j * CH, CH)], iidx.at[j], sem))
    cps.append(pltpu.async_copy(gb_h, gbv, sem))
    for c in cps:
        c.wait()

    urows2 = urows
    irows2 = irows
    gs = []
    for j in range(NCH):
        gs.append(pltpu.async_copy(
            ut_h.at[uidx.at[j]], urows2.at[pl.ds(j * CH, CH)], sem))
        gs.append(pltpu.async_copy(
            it_h.at[iidx.at[j]], irows2.at[pl.ds(j * CH, CH)], sem))
        gs.append(pltpu.async_copy(
            but_h.at[uidx.at[j]], bu.at[pl.ds(j * CH, CH)], sem))
        gs.append(pltpu.async_copy(
            bit_h.at[iidx.at[j]], bi.at[pl.ds(j * CH, CH)], sem))
    for g in gs:
        g.wait()

    gvec = gbv[...]

    def group(g, carry):
        r0 = g * L
        row = r0 + lax.iota(jnp.int32, L)
        acc = jnp.zeros((L,), jnp.float32)
        for d in range(DIM):
            col = jnp.full((L,), d, jnp.int32)
            u = plsc.load_gather(urows, [row, col])
            v = plsc.load_gather(irows, [row, col])
            acc = acc + u * v
        res = acc + gvec + bu[pl.ds(r0, L)] + bi[pl.ds(r0, L)]
        outv[pl.ds(r0, L)] = jnp.clip(res, 1.0, 5.0)
        return carry

    lax.fori_loop(0, BPW // L, group, 0)
    pltpu.sync_copy(outv, out_h.at[pl.ds(base, BPW)])


_mesh = plsc.VectorSubcoreMesh(core_axis_name="c", subcore_axis_name="s")

_svd_sc = functools.partial(
    pl.kernel,
    mesh=_mesh,
    compiler_params=pltpu.CompilerParams(
        needs_layout_passes=False, use_tc_tiling_on_sc=False),
    out_type=jax.ShapeDtypeStruct((B,), jnp.float32),
    scratch_types=[
        pltpu.VMEM((NCH, CH), jnp.int32),
        pltpu.VMEM((NCH, CH), jnp.int32),
        pltpu.VMEM((BPW, DIM), jnp.float32),
        pltpu.VMEM((BPW, DIM), jnp.float32),
        pltpu.VMEM((BPW,), jnp.float32),
        pltpu.VMEM((BPW,), jnp.float32),
        pltpu.VMEM((L,), jnp.float32),
        pltpu.VMEM((BPW,), jnp.float32),
        pltpu.SemaphoreType.DMA,
    ],
)(_body)


@jax.jit
def kernel(user, item, user_table, item_table, bias_user_table,
           bias_item_table, global_bias):
    user = user.astype(jnp.int32)
    item = item.astype(jnp.int32)
    gb = jnp.full((L,), global_bias, jnp.float32)
    out = _svd_sc(user, item, user_table, item_table,
                  bias_user_table.reshape(-1), bias_item_table.reshape(-1),
                  gb)
    return out.reshape(1, B)
